# baseline (speedup 1.0000x reference)
import jax
import jax.numpy as jnp
from jax.experimental import pallas as pl
from jax.experimental.pallas import tpu as pltpu

E = 8
R = 8
SCALE = 1.0

TM = 256


def _moe_lora_kernel(x_ref, wg_ref, eb_ref, wu_ref, bu_ref, acat_ref,
                     bcat_ref, out_ref):
    xb = x_ref[...]
    g = jax.lax.dot_general(xb, wg_ref[...], (((1,), (1,)), ((), ())),
                            preferred_element_type=jnp.float32)
    g = g + eb_ref[...]
    top1 = jnp.argmax(g, axis=-1)[:, None]
    u = jnp.dot(xb, acat_ref[...],
                preferred_element_type=jnp.float32)
    lane = jax.lax.broadcasted_iota(jnp.int32, (TM, E * R), 1) // R
    mask = (lane == top1).astype(jnp.float32)
    u_masked = (u * (mask * SCALE)).astype(jnp.bfloat16)
    base = jax.lax.dot_general(xb.astype(jnp.bfloat16), wu_ref[...],
                               (((1,), (1,)), ((), ())),
                               preferred_element_type=jnp.float32)
    delta = jnp.dot(u_masked, bcat_ref[...],
                    preferred_element_type=jnp.float32)
    out_ref[...] = base + bu_ref[...] + delta


def kernel(x, W_gate, expert_bias, W_up, b_up, A, B):
    Bb, T, H = x.shape
    NT = Bb * T
    x_flat = x.reshape(NT, H)
    W_bf = W_up.astype(jnp.bfloat16)
    A_cat = A.transpose(1, 0, 2).reshape(H, E * R)
    B_cat = B.reshape(E * R, H).astype(jnp.bfloat16)
    eb = expert_bias.reshape(1, E)
    bu = b_up.reshape(1, H)
    TB = NT // TM

    out = pl.pallas_call(
        _moe_lora_kernel,
        grid=(TB,),
        in_specs=[
            pl.BlockSpec((TM, H), lambda t: (t, 0)),
            pl.BlockSpec((E, H), lambda t: (0, 0)),
            pl.BlockSpec((1, E), lambda t: (0, 0)),
            pl.BlockSpec((H, H), lambda t: (0, 0)),
            pl.BlockSpec((1, H), lambda t: (0, 0)),
            pl.BlockSpec((H, E * R), lambda t: (0, 0)),
            pl.BlockSpec((E * R, H), lambda t: (0, 0)),
        ],
        out_specs=pl.BlockSpec((TM, H), lambda t: (t, 0)),
        out_shape=jax.ShapeDtypeStruct((NT, H), jnp.float32),
        compiler_params=pltpu.CompilerParams(
            vmem_limit_bytes=64 * 1024 * 1024),
    )(x_flat, W_gate, eb, W_bf, bu, A_cat, B_cat)
    return out.reshape(Bb, T, H)


# speedup vs baseline: 1.0088x; 1.0088x over previous
import jax
import jax.numpy as jnp
from jax.experimental import pallas as pl
from jax.experimental.pallas import tpu as pltpu

E = 8
R = 8
SCALE = 1.0

TM = 256


def _moe_lora_kernel(x_ref, wg_ref, eb_ref, wu_ref, bu_ref, acat_ref,
                     bcat_ref, out_ref):
    xb = x_ref[...]
    g = jax.lax.dot_general(xb, wg_ref[...], (((1,), (1,)), ((), ())),
                            preferred_element_type=jnp.float32)
    g = g + eb_ref[...]
    top1 = jnp.argmax(g, axis=-1)[:, None]
    u = jnp.dot(xb, acat_ref[...],
                preferred_element_type=jnp.float32)
    lane = jax.lax.broadcasted_iota(jnp.int32, (TM, E * R), 1) // R
    mask = (lane == top1).astype(jnp.float32)
    u_masked = u * (mask * SCALE)
    base = jax.lax.dot_general(xb.astype(jnp.bfloat16), wu_ref[...],
                               (((1,), (1,)), ((), ())),
                               preferred_element_type=jnp.float32)
    delta = jnp.dot(u_masked, bcat_ref[...],
                    preferred_element_type=jnp.float32)
    out_ref[...] = base + bu_ref[...] + delta


def kernel(x, W_gate, expert_bias, W_up, b_up, A, B):
    Bb, T, H = x.shape
    NT = Bb * T
    x_flat = x.reshape(NT, H)
    W_bf = W_up.astype(jnp.bfloat16)
    A_cat = A.transpose(1, 0, 2).reshape(H, E * R)
    B_cat = B.reshape(E * R, H)
    eb = expert_bias.reshape(1, E)
    bu = b_up.reshape(1, H)
    TB = NT // TM

    out = pl.pallas_call(
        _moe_lora_kernel,
        grid=(TB,),
        in_specs=[
            pl.BlockSpec((TM, H), lambda t: (t, 0)),
            pl.BlockSpec((E, H), lambda t: (0, 0)),
            pl.BlockSpec((1, E), lambda t: (0, 0)),
            pl.BlockSpec((H, H), lambda t: (0, 0)),
            pl.BlockSpec((1, H), lambda t: (0, 0)),
            pl.BlockSpec((H, E * R), lambda t: (0, 0)),
            pl.BlockSpec((E * R, H), lambda t: (0, 0)),
        ],
        out_specs=pl.BlockSpec((TM, H), lambda t: (t, 0)),
        out_shape=jax.ShapeDtypeStruct((NT, H), jnp.float32),
        compiler_params=pltpu.CompilerParams(
            vmem_limit_bytes=64 * 1024 * 1024),
    )(x_flat, W_gate, eb, W_bf, bu, A_cat, B_cat)
    return out.reshape(Bb, T, H)


# speedup vs baseline: 1.0923x; 1.0827x over previous
import jax
import jax.numpy as jnp
from jax.experimental import pallas as pl
from jax.experimental.pallas import tpu as pltpu

E = 8
R = 8
SCALE = 1.0

TM = 256


def _moe_lora_kernel(x_ref, wf_ref, wg_ref, eb_ref, bu_ref, acat_ref,
                     bcat_ref, out_ref, wbf_scr, u_scr):
    p = pl.program_id(0)
    t = pl.program_id(1)

    @pl.when(p == 0)
    def _():
        wbf_scr[pl.ds(t * TM, TM), :] = wf_ref[...].astype(jnp.bfloat16)
        xb = x_ref[...]
        g = jax.lax.dot_general(xb, wg_ref[...], (((1,), (1,)), ((), ())),
                                preferred_element_type=jnp.float32)
        g = g + eb_ref[...]
        top1 = jnp.argmax(g, axis=-1)[:, None]
        u = jnp.dot(xb.astype(jnp.bfloat16), acat_ref[...],
                    preferred_element_type=jnp.float32)
        lane = jax.lax.broadcasted_iota(jnp.int32, (TM, E * R), 1) // R
        mask = (lane == top1).astype(jnp.float32)
        u_scr[pl.ds(t * TM, TM), :] = (u * (mask * SCALE)).astype(jnp.bfloat16)

    @pl.when(p == 1)
    def _():
        base = jax.lax.dot_general(
            x_ref[...].astype(jnp.bfloat16), wbf_scr[...],
            (((1,), (1,)), ((), ())),
            preferred_element_type=jnp.float32)
        delta = jnp.dot(u_scr[pl.ds(t * TM, TM), :], bcat_ref[...],
                        preferred_element_type=jnp.float32)
        out_ref[...] = base + bu_ref[...] + delta


def kernel(x, W_gate, expert_bias, W_up, b_up, A, B):
    Bb, T, H = x.shape
    NT = Bb * T
    x_flat = x.reshape(NT, H)
    A_cat = A.transpose(1, 0, 2).reshape(H, E * R).astype(jnp.bfloat16)
    B_cat = B.reshape(E * R, H).astype(jnp.bfloat16)
    eb = expert_bias.reshape(1, E)
    bu = b_up.reshape(1, H)
    TB = NT // TM

    out = pl.pallas_call(
        _moe_lora_kernel,
        grid=(2, TB),
        in_specs=[
            pl.BlockSpec((TM, H), lambda p, t: (t, 0)),
            pl.BlockSpec((TM, H), lambda p, t: (t * (1 - p), 0)),
            pl.BlockSpec((E, H), lambda p, t: (0, 0)),
            pl.BlockSpec((1, E), lambda p, t: (0, 0)),
            pl.BlockSpec((1, H), lambda p, t: (0, 0)),
            pl.BlockSpec((H, E * R), lambda p, t: (0, 0)),
            pl.BlockSpec((E * R, H), lambda p, t: (0, 0)),
        ],
        out_specs=pl.BlockSpec((TM, H), lambda p, t: (t * p, 0)),
        out_shape=jax.ShapeDtypeStruct((NT, H), jnp.float32),
        scratch_shapes=[
            pltpu.VMEM((H, H), jnp.bfloat16),
            pltpu.VMEM((NT, E * R), jnp.bfloat16),
        ],
        compiler_params=pltpu.CompilerParams(
            vmem_limit_bytes=63 * 1024 * 1024),
    )(x_flat, W_up, W_gate, eb, bu, A_cat, B_cat)
    return out.reshape(Bb, T, H)
